# Initial kernel scaffold; baseline (speedup 1.0000x reference)
import jax
import jax.numpy as jnp
from jax.experimental import pallas as pl


def kernel(feat, edge_index, feat1, feat2, W_neigh, W_self, bias):
    raise NotImplementedError("write your pallas kernel here")


# speedup vs baseline: 12.1120x; 12.1120x over previous
import functools

import jax
import jax.numpy as jnp
from jax import lax
from jax.experimental import pallas as pl
from jax.experimental.pallas import tpu as pltpu
from jax.experimental.pallas import tpu_sc as plsc

N_SRC = 50000
N_DST = 50000
E = 1600000
D_HID = 10
D_OUT = 128
DP = 16

NC, NS = 2, 16
SPLIT = 26000
TRASH0 = SPLIT
ROWS_PER_SUBCORE = 1632
N_ACC = ROWS_PER_SUBCORE * NS

CHUNK = 128
BB = 16
NBLK = 50
NB = BB * NBLK
E_PAD = NS * NB * CHUNK
TC_B = 2000
SPLIT_BLKS = SPLIT // TC_B


def _sc_aggregate(feat1p, srcp, dstp, zeros):
    mesh = plsc.VectorSubcoreMesh(core_axis_name="c", subcore_axis_name="s")

    @functools.partial(
        pl.kernel,
        mesh=mesh,
        compiler_params=pltpu.CompilerParams(use_tc_tiling_on_sc=False),
        out_type=pltpu.HBM((NC, N_ACC, DP), jnp.float32),
        scratch_types=[
            pltpu.VMEM((BB, CHUNK), jnp.int32),
            pltpu.VMEM((BB, CHUNK), jnp.int32),
            pltpu.VMEM((BB, CHUNK), jnp.int32),
            pltpu.VMEM((2, CHUNK, DP), jnp.float32),
            pltpu.VMEM_SHARED((N_ACC, DP), jnp.float32),
            pltpu.SemaphoreType.DMA,
            pltpu.SemaphoreType.DMA,
        ],
    )
    def k(feat1_hbm, src_hbm, dst_hbm, zeros_hbm, out_hbm,
          src_v, dst_v, rmp_v, rows_v, acc_sh, sem0, sem1):
        c = lax.axis_index("c")
        s = lax.axis_index("s")
        base = c * SPLIT
        trash = TRASH0 + s

        r0 = s * ROWS_PER_SUBCORE
        pltpu.sync_copy(zeros_hbm, acc_sh.at[pl.ds(r0, ROWS_PER_SUBCORE)])
        plsc.subcore_barrier()

        sems = (sem0, sem1)

        def blk_body(blk, carry):
            b0 = blk * BB
            pltpu.sync_copy(src_hbm.at[s, pl.ds(b0, BB)], src_v)
            pltpu.sync_copy(dst_hbm.at[s, pl.ds(b0, BB)], dst_v)
            for b in range(BB):
                for kk in range(CHUNK // 16):
                    d = dst_v[b, pl.ds(16 * kk, 16)]
                    r = d - base
                    ok = (r >= 0) & (r < SPLIT)
                    rmp_v[b, pl.ds(16 * kk, 16)] = jnp.where(ok, r, trash)
            copies = [pltpu.async_copy(feat1_hbm.at[src_v.at[0]],
                                       rows_v.at[0], sems[0])]
            for b in range(BB):
                if b + 1 < BB:
                    copies.append(
                        pltpu.async_copy(feat1_hbm.at[src_v.at[b + 1]],
                                         rows_v.at[(b + 1) % 2],
                                         sems[(b + 1) % 2]))
                copies[b].wait()
                pltpu.sync_copy(rows_v.at[b % 2], acc_sh.at[rmp_v.at[b]],
                                add=True)
            return carry

        lax.fori_loop(0, NBLK, blk_body, 0)
        plsc.subcore_barrier()
        pltpu.sync_copy(acc_sh.at[pl.ds(r0, ROWS_PER_SUBCORE)],
                        out_hbm.at[c, pl.ds(r0, ROWS_PER_SUBCORE)])

    return k(feat1p, srcp, dstp, zeros)


def _tc_finish(acc2, feat2p, w_cat, bias2):
    grid = N_DST // TC_B

    def body(acc_ref, f2_ref, w_ref, b_ref, o_ref):
        a = acc_ref[0]
        deg = a[:, D_HID:D_HID + 1]
        h = a * (1.0 / jnp.maximum(deg, 1.0))
        x = jnp.concatenate([f2_ref[...], h], axis=1)
        o_ref[...] = (jnp.dot(x, w_ref[...], preferred_element_type=jnp.float32)
                      + b_ref[...])

    def acc_map(i):
        half = (i >= SPLIT_BLKS).astype(jnp.int32)
        return (half, i - half * SPLIT_BLKS, 0)

    return pl.pallas_call(
        body,
        grid=(grid,),
        in_specs=[
            pl.BlockSpec((1, TC_B, DP), acc_map),
            pl.BlockSpec((TC_B, DP), lambda i: (i, 0)),
            pl.BlockSpec((2 * DP, D_OUT), lambda i: (0, 0)),
            pl.BlockSpec((1, D_OUT), lambda i: (0, 0)),
        ],
        out_specs=pl.BlockSpec((TC_B, D_OUT), lambda i: (i, 0)),
        out_shape=jax.ShapeDtypeStruct((N_DST, D_OUT), jnp.float32),
    )(acc2, feat2p, w_cat, bias2)


def kernel(feat, edge_index, feat1, feat2, W_neigh, W_self, bias):
    del feat
    f32 = jnp.float32

    ones = jnp.ones((N_SRC, 1), f32)
    zcols = jnp.zeros((N_SRC, DP - D_HID - 1), f32)
    feat1p = jnp.concatenate([feat1, ones, zcols], axis=1)
    feat1p = jnp.concatenate([feat1p, jnp.zeros((8, DP), f32)], axis=0)

    pad = E_PAD - E
    srcp = jnp.concatenate(
        [edge_index[0], jnp.full((pad,), N_SRC, jnp.int32)]).reshape(NS, NB, CHUNK)
    dstp = jnp.concatenate(
        [edge_index[1], jnp.zeros((pad,), jnp.int32)]).reshape(NS, NB, CHUNK)
    zeros = jnp.zeros((ROWS_PER_SUBCORE, DP), f32)

    acc2 = _sc_aggregate(feat1p, srcp, dstp, zeros)

    feat2p = jnp.concatenate([feat2, jnp.zeros((N_DST, DP - D_HID), f32)], axis=1)
    wpad = jnp.zeros((DP - D_HID, D_OUT), f32)
    w_cat = jnp.concatenate([W_self, wpad, W_neigh, wpad], axis=0)
    bias2 = bias.reshape(1, D_OUT)

    return _tc_finish(acc2, feat2p, w_cat, bias2)


# speedup vs baseline: 14.2043x; 1.1727x over previous
import functools

import jax
import jax.numpy as jnp
from jax import lax
from jax.experimental import pallas as pl
from jax.experimental.pallas import tpu as pltpu
from jax.experimental.pallas import tpu_sc as plsc

N_SRC = 50000
N_DST = 50000
E = 1600000
D_HID = 10
D_OUT = 128
DP = 16

NC, NS = 2, 16
SPLIT = 26000
TRASH0 = SPLIT
ROWS_PER_SUBCORE = 1632
N_ACC = ROWS_PER_SUBCORE * NS

CHUNK = 128
BB = 16
NBLK = 50
NB = BB * NBLK
E_PAD = NS * NB * CHUNK
TC_B = 2000
SPLIT_BLKS = SPLIT // TC_B


def _sc_aggregate(feat1p, srcp, dstp, zeros):
    mesh = plsc.VectorSubcoreMesh(core_axis_name="c", subcore_axis_name="s")

    @functools.partial(
        pl.kernel,
        mesh=mesh,
        compiler_params=pltpu.CompilerParams(use_tc_tiling_on_sc=False),
        out_type=pltpu.HBM((NC, N_ACC, DP), jnp.float32),
        scratch_types=[
            pltpu.VMEM((BB, CHUNK), jnp.int32),
            pltpu.VMEM((BB, CHUNK), jnp.int32),
            pltpu.VMEM((BB, CHUNK), jnp.int32),
            pltpu.VMEM((BB, CHUNK, DP), jnp.float32),
            pltpu.VMEM_SHARED((N_ACC, DP), jnp.float32),
            pltpu.SemaphoreType.DMA,
            pltpu.SemaphoreType.DMA,
        ],
    )
    def k(feat1_hbm, src_hbm, dst_hbm, zeros_hbm, out_hbm,
          src_v, dst_v, rmp_v, rows_v, acc_sh, sem0, sem1):
        c = lax.axis_index("c")
        s = lax.axis_index("s")
        base = c * SPLIT
        trash = TRASH0 + s

        r0 = s * ROWS_PER_SUBCORE
        pltpu.sync_copy(zeros_hbm, acc_sh.at[pl.ds(r0, ROWS_PER_SUBCORE)])
        plsc.subcore_barrier()

        def blk_body(blk, carry):
            b0 = blk * BB
            pltpu.sync_copy(src_hbm.at[s, pl.ds(b0, BB)], src_v)
            pltpu.sync_copy(dst_hbm.at[s, pl.ds(b0, BB)], dst_v)
            for b in range(BB):
                for kk in range(CHUNK // 16):
                    d = dst_v[b, pl.ds(16 * kk, 16)]
                    r = d - base
                    ok = (r >= 0) & (r < SPLIT)
                    rmp_v[b, pl.ds(16 * kk, 16)] = jnp.where(ok, r, trash)
            gcopies = [pltpu.async_copy(feat1_hbm.at[src_v.at[b]],
                                        rows_v.at[b], sem0)
                       for b in range(BB)]
            scopies = []
            for b in range(BB):
                gcopies[b].wait()
                scopies.append(
                    pltpu.async_copy(rows_v.at[b], acc_sh.at[rmp_v.at[b]],
                                     sem1, add=True))
            for cp in scopies:
                cp.wait()
            return carry

        lax.fori_loop(0, NBLK, blk_body, 0)
        plsc.subcore_barrier()
        pltpu.sync_copy(acc_sh.at[pl.ds(r0, ROWS_PER_SUBCORE)],
                        out_hbm.at[c, pl.ds(r0, ROWS_PER_SUBCORE)])

    return k(feat1p, srcp, dstp, zeros)


def _tc_finish(acc2, feat2p, w_cat, bias2):
    grid = N_DST // TC_B

    def body(acc_ref, f2_ref, w_ref, b_ref, o_ref):
        a = acc_ref[0]
        deg = a[:, D_HID:D_HID + 1]
        h = a * (1.0 / jnp.maximum(deg, 1.0))
        x = jnp.concatenate([f2_ref[...], h], axis=1)
        o_ref[...] = (jnp.dot(x, w_ref[...], preferred_element_type=jnp.float32)
                      + b_ref[...])

    def acc_map(i):
        half = (i >= SPLIT_BLKS).astype(jnp.int32)
        return (half, i - half * SPLIT_BLKS, 0)

    return pl.pallas_call(
        body,
        grid=(grid,),
        in_specs=[
            pl.BlockSpec((1, TC_B, DP), acc_map),
            pl.BlockSpec((TC_B, DP), lambda i: (i, 0)),
            pl.BlockSpec((2 * DP, D_OUT), lambda i: (0, 0)),
            pl.BlockSpec((1, D_OUT), lambda i: (0, 0)),
        ],
        out_specs=pl.BlockSpec((TC_B, D_OUT), lambda i: (i, 0)),
        out_shape=jax.ShapeDtypeStruct((N_DST, D_OUT), jnp.float32),
    )(acc2, feat2p, w_cat, bias2)


def kernel(feat, edge_index, feat1, feat2, W_neigh, W_self, bias):
    del feat
    f32 = jnp.float32

    ones = jnp.ones((N_SRC, 1), f32)
    zcols = jnp.zeros((N_SRC, DP - D_HID - 1), f32)
    feat1p = jnp.concatenate([feat1, ones, zcols], axis=1)
    feat1p = jnp.concatenate([feat1p, jnp.zeros((8, DP), f32)], axis=0)

    pad = E_PAD - E
    srcp = jnp.concatenate(
        [edge_index[0], jnp.full((pad,), N_SRC, jnp.int32)]).reshape(NS, NB, CHUNK)
    dstp = jnp.concatenate(
        [edge_index[1], jnp.zeros((pad,), jnp.int32)]).reshape(NS, NB, CHUNK)
    zeros = jnp.zeros((ROWS_PER_SUBCORE, DP), f32)

    acc2 = _sc_aggregate(feat1p, srcp, dstp, zeros)

    feat2p = jnp.concatenate([feat2, jnp.zeros((N_DST, DP - D_HID), f32)], axis=1)
    wpad = jnp.zeros((DP - D_HID, D_OUT), f32)
    w_cat = jnp.concatenate([W_self, wpad, W_neigh, wpad], axis=0)
    bias2 = bias.reshape(1, D_OUT)

    return _tc_finish(acc2, feat2p, w_cat, bias2)


# speedup vs baseline: 15.5069x; 1.0917x over previous
import functools

import jax
import jax.numpy as jnp
from jax import lax
from jax.experimental import pallas as pl
from jax.experimental.pallas import tpu as pltpu
from jax.experimental.pallas import tpu_sc as plsc

N_SRC = 50000
N_DST = 50000
E = 1600000
D_HID = 10
D_OUT = 128
DP = 16

NC, NS = 2, 16
SPLIT = 26000
TRASH0 = SPLIT
ROWS_PER_SUBCORE = 1632
N_ACC = ROWS_PER_SUBCORE * NS

CHUNK = 128
BBC = 16 * CHUNK
NTRASH = 7
BB = 16
NBLK = 50
NB = BB * NBLK
E_PAD = NS * NB * CHUNK
TC_B = 2000
SPLIT_BLKS = SPLIT // TC_B


def _sc_aggregate(feat1p, srcp, dstp, zeros):
    mesh = plsc.VectorSubcoreMesh(core_axis_name="c", subcore_axis_name="s")

    @functools.partial(
        pl.kernel,
        mesh=mesh,
        compiler_params=pltpu.CompilerParams(use_tc_tiling_on_sc=False),
        out_type=pltpu.HBM((NC, N_ACC, DP), jnp.float32),
        scratch_types=[
            pltpu.VMEM((2, BBC), jnp.int32),
            pltpu.VMEM((2, BBC), jnp.int32),
            pltpu.VMEM((2, BBC), jnp.int32),
            pltpu.VMEM((2, BBC, DP), jnp.float32),
            pltpu.VMEM_SHARED((N_ACC, DP), jnp.float32),
            pltpu.SemaphoreType.DMA,
            pltpu.SemaphoreType.DMA,
            pltpu.SemaphoreType.DMA,
            pltpu.SemaphoreType.DMA,
        ],
    )
    def k(feat1_hbm, src_hbm, dst_hbm, zeros_hbm, out_hbm,
          src_v, dst_v, rmp_v, rows_v, acc_sh, gsem, g2sem, ssem, isem):
        c = lax.axis_index("c")
        s = lax.axis_index("s")
        base = c * SPLIT

        r0 = s * ROWS_PER_SUBCORE
        pltpu.sync_copy(zeros_hbm, acc_sh.at[pl.ds(r0, ROWS_PER_SUBCORE)])
        plsc.subcore_barrier()

        def remap(par):
            for kk in range(BBC // 16):
                d = dst_v[par, pl.ds(16 * kk, 16)]
                r = d - base
                ok = (r >= 0) & (r < SPLIT)
                trash = TRASH0 + s * NTRASH + (kk % NTRASH)
                rmp_v[par, pl.ds(16 * kk, 16)] = jnp.where(ok, r, trash)

        def fire_idx(bi, par):
            pltpu.async_copy(src_hbm.at[s, pl.ds(bi * BBC, BBC)],
                             src_v.at[par], isem)
            pltpu.async_copy(dst_hbm.at[s, pl.ds(bi * BBC, BBC)],
                             dst_v.at[par], isem)

        def wait_idx(par):
            pltpu.make_async_copy(src_hbm.at[0, pl.ds(0, BBC)],
                                  src_v.at[par], isem).wait()
            pltpu.make_async_copy(src_hbm.at[0, pl.ds(0, BBC)],
                                  dst_v.at[par], isem).wait()

        H = BBC // 2

        def fire_gather(par):
            pltpu.async_copy(feat1_hbm.at[src_v.at[par, pl.ds(0, H)]],
                             rows_v.at[par, pl.ds(0, H)], gsem)
            pltpu.async_copy(feat1_hbm.at[src_v.at[par, pl.ds(H, H)]],
                             rows_v.at[par, pl.ds(H, H)], g2sem)

        def drain_rows(sem, par):
            pltpu.make_async_copy(feat1_hbm.at[pl.ds(0, BBC)],
                                  rows_v.at[par], sem).wait()

        def fire_scatter(par):
            pltpu.async_copy(rows_v.at[par], acc_sh.at[rmp_v.at[par]],
                             ssem, add=True)

        pltpu.sync_copy(src_hbm.at[s, pl.ds(0, BBC)], src_v.at[0])
        pltpu.sync_copy(dst_hbm.at[s, pl.ds(0, BBC)], dst_v.at[0])
        remap(0)
        fire_gather(0)
        fire_idx(1, 1)

        T = NBLK // 2

        def pair_body(t, carry):
            for off, par in ((0, 0), (1, 1)):
                bi = 2 * t + off
                npar = 1 - par
                if off == 0:
                    @pl.when(t > 0)
                    def _():
                        drain_rows(ssem, npar)
                else:
                    drain_rows(ssem, npar)
                if off == 0:
                    wait_idx(npar)
                    remap(npar)
                    fire_gather(npar)
                else:
                    @pl.when(t < T - 1)
                    def _():
                        wait_idx(npar)
                        remap(npar)
                        fire_gather(npar)
                pltpu.make_async_copy(feat1_hbm.at[pl.ds(0, H)],
                                      rows_v.at[par, pl.ds(0, H)], gsem).wait()
                pltpu.make_async_copy(feat1_hbm.at[pl.ds(0, H)],
                                      rows_v.at[par, pl.ds(H, H)], g2sem).wait()
                fire_scatter(par)
                @pl.when(t < T - 1)
                def _():
                    fire_idx(bi + 2, par)
            return carry

        lax.fori_loop(0, T, pair_body, 0)
        drain_rows(ssem, 1)
        plsc.subcore_barrier()
        pltpu.sync_copy(acc_sh.at[pl.ds(r0, ROWS_PER_SUBCORE)],
                        out_hbm.at[c, pl.ds(r0, ROWS_PER_SUBCORE)])

    return k(feat1p, srcp, dstp, zeros)


def _tc_finish(acc2, feat2p, w_cat, bias2):
    grid = N_DST // TC_B

    def body(acc_ref, f2_ref, w_ref, b_ref, o_ref):
        a = acc_ref[0]
        deg = a[:, D_HID:D_HID + 1]
        h = a * (1.0 / jnp.maximum(deg, 1.0))
        x = jnp.concatenate([f2_ref[...], h], axis=1)
        o_ref[...] = (jnp.dot(x, w_ref[...], preferred_element_type=jnp.float32)
                      + b_ref[...])

    def acc_map(i):
        half = (i >= SPLIT_BLKS).astype(jnp.int32)
        return (half, i - half * SPLIT_BLKS, 0)

    return pl.pallas_call(
        body,
        grid=(grid,),
        in_specs=[
            pl.BlockSpec((1, TC_B, DP), acc_map),
            pl.BlockSpec((TC_B, DP), lambda i: (i, 0)),
            pl.BlockSpec((2 * DP, D_OUT), lambda i: (0, 0)),
            pl.BlockSpec((1, D_OUT), lambda i: (0, 0)),
        ],
        out_specs=pl.BlockSpec((TC_B, D_OUT), lambda i: (i, 0)),
        out_shape=jax.ShapeDtypeStruct((N_DST, D_OUT), jnp.float32),
    )(acc2, feat2p, w_cat, bias2)


def kernel(feat, edge_index, feat1, feat2, W_neigh, W_self, bias):
    del feat
    f32 = jnp.float32

    ones = jnp.ones((N_SRC, 1), f32)
    zcols = jnp.zeros((N_SRC, DP - D_HID - 1), f32)
    feat1p = jnp.concatenate([feat1, ones, zcols], axis=1)
    feat1p = jnp.concatenate([feat1p, jnp.zeros((8, DP), f32)], axis=0)

    pad = E_PAD - E
    srcp = jnp.concatenate(
        [edge_index[0], jnp.full((pad,), N_SRC, jnp.int32)]).reshape(NS, NB * CHUNK)
    dstp = jnp.concatenate(
        [edge_index[1], jnp.zeros((pad,), jnp.int32)]).reshape(NS, NB * CHUNK)
    zeros = jnp.zeros((ROWS_PER_SUBCORE, DP), f32)

    acc2 = _sc_aggregate(feat1p, srcp, dstp, zeros)

    feat2p = jnp.concatenate([feat2, jnp.zeros((N_DST, DP - D_HID), f32)], axis=1)
    wpad = jnp.zeros((DP - D_HID, D_OUT), f32)
    w_cat = jnp.concatenate([W_self, wpad, W_neigh, wpad], axis=0)
    bias2 = bias.reshape(1, D_OUT)

    return _tc_finish(acc2, feat2p, w_cat, bias2)


# speedup vs baseline: 31.0912x; 2.0050x over previous
import functools

import jax
import jax.numpy as jnp
from jax import lax
from jax.experimental import pallas as pl
from jax.experimental.pallas import tpu as pltpu
from jax.experimental.pallas import tpu_sc as plsc

N_SRC = 50000
N_DST = 50000
E = 1600000
D_HID = 10
D_OUT = 128
DP = 16

NC, NS = 2, 16
SPLIT = 26000
TRASH0 = SPLIT
ROWS_PER_SUBCORE = 1632
N_ACC = ROWS_PER_SUBCORE * NS

BBC = 2048
NTRASH = 7
EPT = E // NS
NBLK = EPT // BBC
TAIL = EPT - NBLK * BBC
TC_B = 2000
SPLIT_BLKS = SPLIT // TC_B


def _sc_aggregate(feat1p, edge_index, zeros):
    mesh = plsc.VectorSubcoreMesh(core_axis_name="c", subcore_axis_name="s")

    @functools.partial(
        pl.kernel,
        mesh=mesh,
        compiler_params=pltpu.CompilerParams(use_tc_tiling_on_sc=False),
        out_type=pltpu.HBM((NC, N_ACC, DP), jnp.float32),
        scratch_types=[
            pltpu.VMEM((2, BBC), jnp.int32),
            pltpu.VMEM((2, BBC), jnp.int32),
            pltpu.VMEM((2, BBC), jnp.int32),
            pltpu.VMEM((2, BBC, DP), jnp.float32),
            pltpu.VMEM_SHARED((N_ACC, DP), jnp.float32),
            pltpu.SemaphoreType.DMA,
            pltpu.SemaphoreType.DMA,
            pltpu.SemaphoreType.DMA,
        ],
    )
    def k(feat1_hbm, ei_hbm, zeros_hbm, out_hbm,
          src_v, dst_v, rmp_v, rows_v, acc_sh, gsem, ssem, isem):
        c = lax.axis_index("c")
        s = lax.axis_index("s")
        base = c * SPLIT

        r0 = s * ROWS_PER_SUBCORE
        pltpu.sync_copy(zeros_hbm, acc_sh.at[pl.ds(r0, ROWS_PER_SUBCORE)])
        plsc.subcore_barrier()

        def remap(par):
            for kk in range(BBC // 16):
                d = dst_v[par, pl.ds(16 * kk, 16)]
                r = d - base
                ok = (r >= 0) & (r < SPLIT)
                trash = TRASH0 + s * NTRASH + (kk % NTRASH)
                rmp_v[par, pl.ds(16 * kk, 16)] = jnp.where(ok, r, trash)

        e0 = s * EPT

        def fire_idx(bi, par):
            pltpu.async_copy(ei_hbm.at[0, pl.ds(e0 + bi * BBC, BBC)],
                             src_v.at[par], isem)
            pltpu.async_copy(ei_hbm.at[1, pl.ds(e0 + bi * BBC, BBC)],
                             dst_v.at[par], isem)

        def wait_idx(par):
            pltpu.make_async_copy(ei_hbm.at[0, pl.ds(0, BBC)],
                                  src_v.at[par], isem).wait()
            pltpu.make_async_copy(ei_hbm.at[0, pl.ds(0, BBC)],
                                  dst_v.at[par], isem).wait()

        def fire_gather(par):
            pltpu.async_copy(feat1_hbm.at[src_v.at[par]], rows_v.at[par], gsem)

        def drain_rows(sem, par):
            pltpu.make_async_copy(feat1_hbm.at[pl.ds(0, BBC)],
                                  rows_v.at[par], sem).wait()

        def fire_scatter(par):
            pltpu.async_copy(rows_v.at[par], acc_sh.at[rmp_v.at[par]],
                             ssem, add=True)

        pltpu.sync_copy(ei_hbm.at[0, pl.ds(e0, BBC)], src_v.at[0])
        pltpu.sync_copy(ei_hbm.at[1, pl.ds(e0, BBC)], dst_v.at[0])
        remap(0)
        fire_gather(0)
        fire_idx(1, 1)

        T = NBLK // 2

        def pair_body(t, carry):
            for off, par in ((0, 0), (1, 1)):
                bi = 2 * t + off
                npar = 1 - par
                if off == 0:
                    @pl.when(t > 0)
                    def _():
                        drain_rows(ssem, npar)
                else:
                    drain_rows(ssem, npar)
                if off == 0:
                    wait_idx(npar)
                    remap(npar)
                    fire_gather(npar)
                else:
                    @pl.when(t < T - 1)
                    def _():
                        wait_idx(npar)
                        remap(npar)
                        fire_gather(npar)
                drain_rows(gsem, par)
                fire_scatter(par)
                @pl.when(t < T - 1)
                def _():
                    fire_idx(bi + 2, par)
            return carry

        lax.fori_loop(0, T, pair_body, 0)
        drain_rows(ssem, 1)
        t0 = e0 + NBLK * BBC
        pltpu.sync_copy(ei_hbm.at[0, pl.ds(t0, TAIL)],
                        src_v.at[0, pl.ds(0, TAIL)])
        pltpu.sync_copy(ei_hbm.at[1, pl.ds(t0, TAIL)],
                        dst_v.at[0, pl.ds(0, TAIL)])
        for kk in range(TAIL // 16):
            d = dst_v[0, pl.ds(16 * kk, 16)]
            r = d - base
            ok = (r >= 0) & (r < SPLIT)
            trash = TRASH0 + s * NTRASH + (kk % NTRASH)
            rmp_v[0, pl.ds(16 * kk, 16)] = jnp.where(ok, r, trash)
        zeros16 = jnp.zeros((16,), jnp.int32)
        for kk in range(TAIL // 16, BBC // 16):
            src_v[0, pl.ds(16 * kk, 16)] = zeros16
            rmp_v[0, pl.ds(16 * kk, 16)] = zeros16 + (TRASH0 + s * NTRASH)
        fire_gather(0)
        drain_rows(gsem, 0)
        fire_scatter(0)
        drain_rows(ssem, 0)
        plsc.subcore_barrier()
        pltpu.sync_copy(acc_sh.at[pl.ds(r0, ROWS_PER_SUBCORE)],
                        out_hbm.at[c, pl.ds(r0, ROWS_PER_SUBCORE)])

    return k(feat1p, edge_index, zeros)


def _tc_finish(acc2, feat2, w_self, w_neigh_p, bias2):
    grid = N_DST // TC_B

    def body(acc_ref, f2_ref, ws_ref, wn_ref, b_ref, o_ref):
        a = acc_ref[0]
        deg = a[:, D_HID:D_HID + 1]
        h = a * (1.0 / jnp.maximum(deg, 1.0))
        o_ref[...] = (
            jnp.dot(f2_ref[...], ws_ref[...], preferred_element_type=jnp.float32)
            + jnp.dot(h, wn_ref[...], preferred_element_type=jnp.float32)
            + b_ref[...])

    def acc_map(i):
        half = (i >= SPLIT_BLKS).astype(jnp.int32)
        return (half, i - half * SPLIT_BLKS, 0)

    return pl.pallas_call(
        body,
        grid=(grid,),
        in_specs=[
            pl.BlockSpec((1, TC_B, DP), acc_map),
            pl.BlockSpec((TC_B, D_HID), lambda i: (i, 0)),
            pl.BlockSpec((D_HID, D_OUT), lambda i: (0, 0)),
            pl.BlockSpec((DP, D_OUT), lambda i: (0, 0)),
            pl.BlockSpec((1, D_OUT), lambda i: (0, 0)),
        ],
        out_specs=pl.BlockSpec((TC_B, D_OUT), lambda i: (i, 0)),
        out_shape=jax.ShapeDtypeStruct((N_DST, D_OUT), jnp.float32),
    )(acc2, feat2, w_self, w_neigh_p, bias2)


def kernel(feat, edge_index, feat1, feat2, W_neigh, W_self, bias):
    del feat
    f32 = jnp.float32

    ones = jnp.ones((N_SRC, 1), f32)
    zcols = jnp.zeros((N_SRC, DP - D_HID - 1), f32)
    feat1p = jnp.concatenate([feat1, ones, zcols], axis=1)
    feat1p = jnp.concatenate([feat1p, jnp.zeros((8, DP), f32)], axis=0)

    zeros = jnp.zeros((ROWS_PER_SUBCORE, DP), f32)

    acc2 = _sc_aggregate(feat1p, edge_index, zeros)

    w_neigh_p = jnp.concatenate(
        [W_neigh, jnp.zeros((DP - D_HID, D_OUT), f32)], axis=0)
    bias2 = bias.reshape(1, D_OUT)

    return _tc_finish(acc2, feat2, W_self, w_neigh_p, bias2)
